# baseline (speedup 1.0000x reference)
import functools

import jax
import jax.numpy as jnp
from jax import lax
from jax.experimental import pallas as pl
from jax.experimental.pallas import tpu as pltpu
from jax.experimental.pallas import tpu_sc as plsc

NODE = 128
NW = 32
CHUNK = 128
CHUNKS_PER_W = 25
PER_W = CHUNK * CHUNKS_PER_W
B_PAD = NW * PER_W

_mesh = plsc.VectorSubcoreMesh(core_axis_name="c", subcore_axis_name="s")


@functools.partial(
    pl.kernel,
    mesh=_mesh,
    out_type=jax.ShapeDtypeStruct((NW, CHUNKS_PER_W, CHUNK, NODE), jnp.float32),
    scratch_types=[
        pltpu.VMEM((CHUNKS_PER_W, CHUNK), jnp.int32),
        pltpu.VMEM((CHUNK, NODE), jnp.float32),
        pltpu.SemaphoreType.DMA,
    ],
)
def _embed_lookup(table_hbm, z_hbm, out_hbm, idx_v, rows_v, gsem):
    wid = lax.axis_index("s") * 2 + lax.axis_index("c")
    pltpu.sync_copy(z_hbm.at[wid], idx_v)

    def body(j, carry):
        pltpu.async_copy(table_hbm.at[idx_v.at[j]], rows_v, gsem).wait()
        pltpu.sync_copy(rows_v, out_hbm.at[wid, j])
        return carry

    lax.fori_loop(0, CHUNKS_PER_W, body, 0)


def kernel(Z, table):
    z_pad = jnp.pad(Z.astype(jnp.int32), (0, B_PAD - Z.shape[0]))
    z3 = z_pad.reshape(NW, CHUNKS_PER_W, CHUNK)
    out = _embed_lookup(table, z3)
    return out.reshape(B_PAD, NODE)[: Z.shape[0]]


# speedup vs baseline: 1.0482x; 1.0482x over previous
import functools

import jax
import jax.numpy as jnp
from jax import lax
from jax.experimental import pallas as pl
from jax.experimental.pallas import tpu as pltpu
from jax.experimental.pallas import tpu_sc as plsc

NODE = 128
NW = 32
CHUNK = 128
CHUNKS_PER_W = 25
PER_W = CHUNK * CHUNKS_PER_W
B_PAD = NW * PER_W

NSLOT = 6
LAG = 3

_mesh = plsc.VectorSubcoreMesh(core_axis_name="c", subcore_axis_name="s")


@functools.partial(
    pl.kernel,
    mesh=_mesh,
    out_type=jax.ShapeDtypeStruct((NW, CHUNKS_PER_W, CHUNK, NODE), jnp.float32),
    scratch_types=[
        pltpu.VMEM((CHUNKS_PER_W, CHUNK), jnp.int32),
        pltpu.VMEM((NSLOT, CHUNK, NODE), jnp.float32),
        pltpu.SemaphoreType.DMA((NSLOT,)),
        pltpu.SemaphoreType.DMA((NSLOT,)),
    ],
)
def _embed_lookup(table_hbm, z_hbm, out_hbm, idx_v, bufs, gsem, ssem):
    wid = lax.axis_index("s") * 2 + lax.axis_index("c")
    pltpu.sync_copy(z_hbm.at[wid], idx_v)

    gathers = {}
    scatters = {}
    for t in range(CHUNKS_PER_W + LAG):
        if t < CHUNKS_PER_W:
            b = t % NSLOT
            if t >= NSLOT:
                scatters[t - NSLOT].wait()
            gathers[t] = pltpu.async_copy(
                table_hbm.at[idx_v.at[t]], bufs.at[b], gsem.at[b]
            )
        i = t - LAG
        if i >= 0:
            b = i % NSLOT
            gathers[i].wait()
            scatters[i] = pltpu.async_copy(
                bufs.at[b], out_hbm.at[wid, i], ssem.at[b]
            )
    for i in range(CHUNKS_PER_W - NSLOT, CHUNKS_PER_W):
        scatters[i].wait()


def kernel(Z, table):
    z_pad = jnp.pad(Z.astype(jnp.int32), (0, B_PAD - Z.shape[0]))
    z3 = z_pad.reshape(NW, CHUNKS_PER_W, CHUNK)
    out = _embed_lookup(table, z3)
    return out.reshape(B_PAD, NODE)[: Z.shape[0]]
